# baseline (speedup 1.0000x reference)
import functools
import math

import jax
import jax.numpy as jnp
from jax import lax
from jax.experimental import pallas as pl
from jax.experimental.pallas import tpu as pltpu
from jax.experimental.pallas import tpu_sc as plsc

D_MODEL = 128
MAX_LEN = 2048

_NUM_CORES = 2
_NUM_SUBCORES = 16
_NUM_WORKERS = _NUM_CORES * _NUM_SUBCORES

_NBUF = 5
_LOOKAHEAD = 2
_CHUNK = 128

_TWO_PI = float(jnp.float32(2 * math.pi))
_PI = float(jnp.float32(math.pi))

_SIN_COEFFS = (
    -1.0, 0.1666666567325592, -0.008333314210176468, 0.00019840311142615974,
    -2.7532287276699208e-06, 2.4701575895846872e-08, -1.3533152376510316e-10,
)
_COS_COEFFS = (
    -1.0, 0.49999991059303284, -0.04166652262210846, 0.0013887969544157386,
    -2.477341695339419e-05, 2.7113293299407815e-07, -1.736882859759703e-09,
)


_INV_TWO_PI = float(jnp.float32(1.0 / (2 * math.pi)))


def _sin_cos_select(x, even_lane):
    k = (x * jnp.float32(_INV_TWO_PI)).astype(jnp.int32).astype(jnp.float32)
    u = (x - k * jnp.float32(_TWO_PI)) - jnp.float32(_PI)
    s = u * u
    ps = jnp.float32(_SIN_COEFFS[-1])
    for c in _SIN_COEFFS[-2::-1]:
        ps = ps * s + jnp.float32(c)
    sinv = ps * u
    pc = jnp.float32(_COS_COEFFS[-1])
    for c in _COS_COEFFS[-2::-1]:
        pc = pc * s + jnp.float32(c)
    return jnp.where(even_lane, sinv, pc)


def _make_kernel(batch):
    b_per_w = batch // _NUM_WORKERS
    n_chunks = b_per_w // _CHUNK
    assert n_chunks % _NBUF == 0 and n_chunks >= 2 * _NBUF
    rows_per_tile = MAX_LEN // _NUM_SUBCORES
    assert rows_per_tile == _CHUNK
    mesh = plsc.VectorSubcoreMesh(core_axis_name="c", subcore_axis_name="s")

    @functools.partial(
        pl.kernel,
        mesh=mesh,
        out_type=jax.ShapeDtypeStruct((batch, D_MODEL), jnp.float32),
        scratch_types=[
            pltpu.VMEM((b_per_w,), jnp.int32),
            pltpu.VMEM((_NBUF, _CHUNK, D_MODEL), jnp.float32),
            pltpu.VMEM((D_MODEL,), jnp.float32),
            pltpu.VMEM_SHARED((MAX_LEN, D_MODEL), jnp.float32),
            pltpu.SemaphoreType.DMA,
            [pltpu.SemaphoreType.DMA] * _NBUF,
            [pltpu.SemaphoreType.DMA] * _NBUF,
        ],
    )
    def pe_lookup(div_hbm, idx_hbm, out_hbm, idx_v, rows_v, div_v, table_sp,
                  isem, gsems, osems):
        sid = lax.axis_index("s")
        wid = sid * _NUM_CORES + lax.axis_index("c")
        base = wid * b_per_w

        idx_load = pltpu.make_async_copy(
            idx_hbm.at[pl.ds(base, b_per_w)], idx_v, isem)
        idx_load.start()

        pltpu.sync_copy(div_hbm, div_v)
        even_lane = lax.rem(lax.iota(jnp.int32, 16), 2) == 0
        row_base = sid * rows_per_tile
        stage = rows_v.at[0]

        def row_body(r, carry):
            rf = (row_base + r).astype(jnp.float32)
            rv = jnp.full((16,), rf, jnp.float32)
            for c in range(D_MODEL // 16):
                d = div_v[pl.ds(16 * c, 16)]
                stage[r, pl.ds(16 * c, 16)] = _sin_cos_select(rv * d, even_lane)
            return carry

        lax.fori_loop(0, rows_per_tile, row_body, 0)
        pltpu.sync_copy(stage, table_sp.at[pl.ds(row_base, rows_per_tile)])
        idx_load.wait()
        plsc.subcore_barrier()

        def gather_desc(j, b):
            return pltpu.make_async_copy(
                table_sp.at[idx_v.at[pl.ds(j * _CHUNK, _CHUNK)]],
                rows_v.at[b], gsems[b])

        def out_desc(j, b):
            return pltpu.make_async_copy(
                rows_v.at[b], out_hbm.at[pl.ds(base + j * _CHUNK, _CHUNK)],
                osems[b])

        for j in range(_LOOKAHEAD):
            gather_desc(j, j % _NBUF).start()

        def step(j, jd, b, bd):
            @pl.when(jd >= _NBUF)
            def _():
                out_desc(jd - _NBUF, bd).wait()

            @pl.when(jd < n_chunks)
            def _():
                gather_desc(jd, bd).start()

            gather_desc(j, b).wait()
            out_desc(j, b).start()

        def body(j2, carry):
            for u in range(_NBUF):
                j = j2 * _NBUF + u
                jd = j + _LOOKAHEAD
                step(j, jd, u, (u + _LOOKAHEAD) % _NBUF)
            return carry

        lax.fori_loop(0, n_chunks // _NBUF, body, 0)

        for j in range(n_chunks - _NBUF + _LOOKAHEAD, n_chunks):
            out_desc(j, j % _NBUF).wait()

    return pe_lookup


def kernel(position, div_term):
    div_full = jnp.repeat(div_term, 2)
    idx = position.reshape(-1)
    batch = idx.shape[0]
    return _make_kernel(batch)(div_full, idx)


# speedup vs baseline: 1.1075x; 1.1075x over previous
import functools
import math

import jax
import jax.numpy as jnp
from jax import lax
from jax.experimental import pallas as pl
from jax.experimental.pallas import tpu as pltpu
from jax.experimental.pallas import tpu_sc as plsc

D_MODEL = 128
MAX_LEN = 2048

_NUM_CORES = 2
_NUM_SUBCORES = 16
_NUM_WORKERS = _NUM_CORES * _NUM_SUBCORES

_NBUF = 5
_LOOKAHEAD = 2
_CHUNK = 128

_TWO_PI = float(jnp.float32(2 * math.pi))
_PI = float(jnp.float32(math.pi))

_SIN_COEFFS = (
    -1.0, 0.1666666567325592, -0.008333314210176468, 0.00019840311142615974,
    -2.7532287276699208e-06, 2.4701575895846872e-08, -1.3533152376510316e-10,
)
_COS_COEFFS = (
    -1.0, 0.49999991059303284, -0.04166652262210846, 0.0013887969544157386,
    -2.477341695339419e-05, 2.7113293299407815e-07, -1.736882859759703e-09,
)


_INV_TWO_PI = float(jnp.float32(1.0 / (2 * math.pi)))


def _sin_cos_select(x, even_lane):
    k = (x * jnp.float32(_INV_TWO_PI)).astype(jnp.int32).astype(jnp.float32)
    u = (x - k * jnp.float32(_TWO_PI)) - jnp.float32(_PI)
    s = u * u
    ps = jnp.float32(_SIN_COEFFS[-1])
    for c in _SIN_COEFFS[-2::-1]:
        ps = ps * s + jnp.float32(c)
    sinv = ps * u
    pc = jnp.float32(_COS_COEFFS[-1])
    for c in _COS_COEFFS[-2::-1]:
        pc = pc * s + jnp.float32(c)
    return jnp.where(even_lane, sinv, pc)


def _make_kernel(batch):
    b_per_w = batch // _NUM_WORKERS
    n_chunks = b_per_w // _CHUNK
    assert n_chunks % _NBUF == 0 and n_chunks >= 2 * _NBUF
    rows_per_tile = MAX_LEN // _NUM_SUBCORES
    assert rows_per_tile == _CHUNK
    mesh = plsc.VectorSubcoreMesh(core_axis_name="c", subcore_axis_name="s")

    @functools.partial(
        pl.kernel,
        mesh=mesh,
        out_type=jax.ShapeDtypeStruct((batch, D_MODEL), jnp.float32),
        scratch_types=[
            pltpu.VMEM((b_per_w,), jnp.int32),
            pltpu.VMEM((_NBUF, _CHUNK, D_MODEL), jnp.float32),
            pltpu.VMEM((D_MODEL,), jnp.float32),
            pltpu.VMEM_SHARED((MAX_LEN, D_MODEL), jnp.float32),
            pltpu.SemaphoreType.DMA,
            [pltpu.SemaphoreType.DMA] * _NBUF,
            [pltpu.SemaphoreType.DMA] * _NBUF,
        ],
    )
    def pe_lookup(div_hbm, idx_hbm, out_hbm, idx_v, rows_v, div_v, table_sp,
                  isem, gsems, osems):
        sid = lax.axis_index("s")
        wid = sid * _NUM_CORES + lax.axis_index("c")
        base = wid * b_per_w

        idx_load = pltpu.make_async_copy(
            idx_hbm.at[pl.ds(base, b_per_w)], idx_v, isem)
        idx_load.start()

        pltpu.sync_copy(div_hbm, div_v)
        even_lane = lax.rem(lax.iota(jnp.int32, 16), 2) == 0
        row_base = sid * rows_per_tile
        stage = rows_v.at[0]

        @plsc.parallel_loop(0, rows_per_tile, unroll=4)
        def row_body(r):
            rf = (row_base + r).astype(jnp.float32)
            rv = jnp.full((16,), rf, jnp.float32)
            for c in range(D_MODEL // 16):
                d = div_v[pl.ds(16 * c, 16)]
                stage[r, pl.ds(16 * c, 16)] = _sin_cos_select(rv * d, even_lane)
        pltpu.sync_copy(stage, table_sp.at[pl.ds(row_base, rows_per_tile)])
        idx_load.wait()
        plsc.subcore_barrier()

        def gather_desc(j, b):
            return pltpu.make_async_copy(
                table_sp.at[idx_v.at[pl.ds(j * _CHUNK, _CHUNK)]],
                rows_v.at[b], gsems[b])

        def out_desc(j, b):
            return pltpu.make_async_copy(
                rows_v.at[b], out_hbm.at[pl.ds(base + j * _CHUNK, _CHUNK)],
                osems[b])

        for j in range(_LOOKAHEAD):
            gather_desc(j, j % _NBUF).start()

        def step(j, jd, b, bd):
            @pl.when(jd >= _NBUF)
            def _():
                out_desc(jd - _NBUF, bd).wait()

            @pl.when(jd < n_chunks)
            def _():
                gather_desc(jd, bd).start()

            gather_desc(j, b).wait()
            out_desc(j, b).start()

        def body(j2, carry):
            for u in range(_NBUF):
                j = j2 * _NBUF + u
                jd = j + _LOOKAHEAD
                step(j, jd, u, (u + _LOOKAHEAD) % _NBUF)
            return carry

        lax.fori_loop(0, n_chunks // _NBUF, body, 0)

        for j in range(n_chunks - _NBUF + _LOOKAHEAD, n_chunks):
            out_desc(j, j % _NBUF).wait()

    return pe_lookup


def kernel(position, div_term):
    div_full = jnp.repeat(div_term, 2)
    idx = position.reshape(-1)
    batch = idx.shape[0]
    return _make_kernel(batch)(div_full, idx)


# speedup vs baseline: 1.1076x; 1.0001x over previous
import functools
import math

import jax
import jax.numpy as jnp
from jax import lax
from jax.experimental import pallas as pl
from jax.experimental.pallas import tpu as pltpu
from jax.experimental.pallas import tpu_sc as plsc

D_MODEL = 128
MAX_LEN = 2048

_NUM_CORES = 2
_NUM_SUBCORES = 16
_NUM_WORKERS = _NUM_CORES * _NUM_SUBCORES

_NBUF = 5
_LOOKAHEAD = 2
_CHUNK = 128

_TWO_PI = float(jnp.float32(2 * math.pi))
_PI = float(jnp.float32(math.pi))

_SIN_COEFFS = (
    -1.0, 0.1666666567325592, -0.008333314210176468, 0.00019840311142615974,
    -2.7532287276699208e-06, 2.4701575895846872e-08, -1.3533152376510316e-10,
)
_COS_COEFFS = (
    -1.0, 0.49999991059303284, -0.04166652262210846, 0.0013887969544157386,
    -2.477341695339419e-05, 2.7113293299407815e-07, -1.736882859759703e-09,
)


_INV_TWO_PI = float(jnp.float32(1.0 / (2 * math.pi)))


def _sin_cos_select(x, even_lane):
    k = (x * jnp.float32(_INV_TWO_PI)).astype(jnp.int32).astype(jnp.float32)
    u = (x - k * jnp.float32(_TWO_PI)) - jnp.float32(_PI)
    s = u * u
    ps = jnp.float32(_SIN_COEFFS[-1])
    for c in _SIN_COEFFS[-2::-1]:
        ps = ps * s + jnp.float32(c)
    sinv = ps * u
    pc = jnp.float32(_COS_COEFFS[-1])
    for c in _COS_COEFFS[-2::-1]:
        pc = pc * s + jnp.float32(c)
    return jnp.where(even_lane, sinv, pc)


def _make_kernel(batch):
    b_per_w = batch // _NUM_WORKERS
    n_chunks = b_per_w // _CHUNK
    assert n_chunks % _NBUF == 0 and n_chunks >= 2 * _NBUF
    rows_per_tile = MAX_LEN // _NUM_SUBCORES
    assert rows_per_tile == _CHUNK
    mesh = plsc.VectorSubcoreMesh(core_axis_name="c", subcore_axis_name="s")

    @functools.partial(
        pl.kernel,
        mesh=mesh,
        out_type=jax.ShapeDtypeStruct((batch, D_MODEL), jnp.float32),
        scratch_types=[
            pltpu.VMEM((b_per_w,), jnp.int32),
            pltpu.VMEM((_NBUF, _CHUNK, D_MODEL), jnp.float32),
            pltpu.VMEM((D_MODEL,), jnp.float32),
            pltpu.VMEM_SHARED((MAX_LEN, D_MODEL), jnp.float32),
            pltpu.SemaphoreType.DMA,
            [pltpu.SemaphoreType.DMA] * _NBUF,
            [pltpu.SemaphoreType.DMA] * _NBUF,
        ],
    )
    def pe_lookup(div_hbm, idx_hbm, out_hbm, idx_v, rows_v, div_v, table_sp,
                  isem, gsems, osems):
        sid = lax.axis_index("s")
        wid = sid * _NUM_CORES + lax.axis_index("c")
        base = wid * b_per_w

        idx_load = pltpu.make_async_copy(
            idx_hbm.at[pl.ds(base, b_per_w)], idx_v, isem)
        idx_load.start()

        pltpu.sync_copy(div_hbm, div_v)
        even_lane = lax.rem(lax.iota(jnp.int32, 16), 2) == 0
        row_base = sid * rows_per_tile
        stage = rows_v.at[0]

        @plsc.parallel_loop(0, rows_per_tile, unroll=8)
        def row_body(r):
            rf = (row_base + r).astype(jnp.float32)
            rv = jnp.full((16,), rf, jnp.float32)
            for c in range(D_MODEL // 16):
                d = div_v[pl.ds(16 * c, 16)]
                stage[r, pl.ds(16 * c, 16)] = _sin_cos_select(rv * d, even_lane)
        pltpu.sync_copy(stage, table_sp.at[pl.ds(row_base, rows_per_tile)])
        idx_load.wait()
        plsc.subcore_barrier()

        def gather_desc(j, b):
            return pltpu.make_async_copy(
                table_sp.at[idx_v.at[pl.ds(j * _CHUNK, _CHUNK)]],
                rows_v.at[b], gsems[b])

        def out_desc(j, b):
            return pltpu.make_async_copy(
                rows_v.at[b], out_hbm.at[pl.ds(base + j * _CHUNK, _CHUNK)],
                osems[b])

        for j in range(_LOOKAHEAD):
            gather_desc(j, j % _NBUF).start()

        def step(j, jd, b, bd):
            @pl.when(jd >= _NBUF)
            def _():
                out_desc(jd - _NBUF, bd).wait()

            @pl.when(jd < n_chunks)
            def _():
                gather_desc(jd, bd).start()

            gather_desc(j, b).wait()
            out_desc(j, b).start()

        def body(j2, carry):
            for u in range(_NBUF):
                j = j2 * _NBUF + u
                jd = j + _LOOKAHEAD
                step(j, jd, u, (u + _LOOKAHEAD) % _NBUF)
            return carry

        lax.fori_loop(0, n_chunks // _NBUF, body, 0)

        for j in range(n_chunks - _NBUF + _LOOKAHEAD, n_chunks):
            out_desc(j, j % _NBUF).wait()

    return pe_lookup


def kernel(position, div_term):
    div_full = jnp.repeat(div_term, 2)
    idx = position.reshape(-1)
    batch = idx.shape[0]
    return _make_kernel(batch)(div_full, idx)


# speedup vs baseline: 1.1278x; 1.0182x over previous
import functools
import math

import jax
import jax.numpy as jnp
from jax import lax
from jax.experimental import pallas as pl
from jax.experimental.pallas import tpu as pltpu
from jax.experimental.pallas import tpu_sc as plsc

D_MODEL = 128
MAX_LEN = 2048

_NUM_CORES = 2
_NUM_SUBCORES = 16
_NUM_WORKERS = _NUM_CORES * _NUM_SUBCORES

_NBUF = 5
_LOOKAHEAD = 2


def _table_body(div_full_ref, out_ref):
    pos = lax.broadcasted_iota(jnp.int32, (MAX_LEN, D_MODEL), 0).astype(jnp.float32)
    angles = pos * div_full_ref[...]
    lane = lax.broadcasted_iota(jnp.int32, (MAX_LEN, D_MODEL), 1)
    out_ref[...] = jnp.where(lane % 2 == 0, jnp.sin(angles), jnp.cos(angles))


def _build_table(div_term):
    div_full = jnp.repeat(div_term, 2).reshape(1, D_MODEL)
    return pl.pallas_call(
        _table_body,
        out_shape=jax.ShapeDtypeStruct((MAX_LEN, D_MODEL), jnp.float32),
    )(div_full)


def _make_gather(batch, chunk):
    b_per_w = batch // _NUM_WORKERS
    n_chunks = b_per_w // chunk
    assert n_chunks % _NBUF == 0 and n_chunks >= 2 * _NBUF
    mesh = plsc.VectorSubcoreMesh(core_axis_name="c", subcore_axis_name="s")

    @functools.partial(
        pl.kernel,
        mesh=mesh,
        out_type=jax.ShapeDtypeStruct((batch, D_MODEL), jnp.float32),
        scratch_types=[
            pltpu.VMEM((b_per_w,), jnp.int32),
            pltpu.VMEM((_NBUF, chunk, D_MODEL), jnp.float32),
            pltpu.VMEM_SHARED((MAX_LEN, D_MODEL), jnp.float32),
            [pltpu.SemaphoreType.DMA] * _NBUF,
            [pltpu.SemaphoreType.DMA] * _NBUF,
        ],
    )
    def gather(table_hbm, idx_hbm, out_hbm, idx_v, rows_v, table_sp,
               gsems, osems):
        wid = lax.axis_index("s") * _NUM_CORES + lax.axis_index("c")
        base = wid * b_per_w

        @pl.when(lax.axis_index("s") == 0)
        def _():
            pltpu.sync_copy(table_hbm, table_sp)

        pltpu.sync_copy(idx_hbm.at[pl.ds(base, b_per_w)], idx_v)
        plsc.subcore_barrier()

        def gather_desc(j, b):
            return pltpu.make_async_copy(
                table_sp.at[idx_v.at[pl.ds(j * chunk, chunk)]],
                rows_v.at[b], gsems[b])

        def out_desc(j, b):
            return pltpu.make_async_copy(
                rows_v.at[b], out_hbm.at[pl.ds(base + j * chunk, chunk)],
                osems[b])

        for j in range(_LOOKAHEAD):
            gather_desc(j, j % _NBUF).start()

        def step(j, jd, b, bd):
            @pl.when(jd >= _NBUF)
            def _():
                out_desc(jd - _NBUF, bd).wait()

            @pl.when(jd < n_chunks)
            def _():
                gather_desc(jd, bd).start()

            gather_desc(j, b).wait()
            out_desc(j, b).start()

        def body(j2, carry):
            for u in range(_NBUF):
                j = j2 * _NBUF + u
                jd = j + _LOOKAHEAD
                step(j, jd, u, (u + _LOOKAHEAD) % _NBUF)
            return carry

        lax.fori_loop(0, n_chunks // _NBUF, body, 0)

        for j in range(n_chunks - _NBUF + _LOOKAHEAD, n_chunks):
            out_desc(j, j % _NBUF).wait()

    return gather


def kernel(position, div_term):
    table = _build_table(div_term)
    idx = position.reshape(-1)
    batch = idx.shape[0]
    gather = _make_gather(batch, chunk=128)
    return gather(table, idx)
